# Initial kernel scaffold; baseline (speedup 1.0000x reference)
import jax
import jax.numpy as jnp
from jax.experimental import pallas as pl


def kernel(numbers, d_ij, weight):
    raise NotImplementedError("write your pallas kernel here")


# speedup vs baseline: 1.0381x; 1.0381x over previous
import functools

import jax
import jax.numpy as jnp
from jax import lax
from jax.experimental import pallas as pl
from jax.experimental.pallas import tpu as pltpu
from jax.experimental.pallas import tpu_sc as plsc

_B = 64
_N = 512
_TOT = _B * _N
_NW = 32
_PER_W = _TOT // _NW
_LANES = 16
_TAB = 128


def _sc_gather_body(nums_hbm, a_tab_hbm, z_tab_hbm, a_out_hbm, z_out_hbm,
                    idx_v, a_v, z_v, sem):
    wid = lax.axis_index("s") * 2 + lax.axis_index("c")
    base = wid * _PER_W
    pltpu.sync_copy(nums_hbm.at[pl.ds(base, _PER_W)], idx_v)
    pltpu.async_copy(a_tab_hbm.at[idx_v], a_v, sem).wait()
    pltpu.async_copy(z_tab_hbm.at[idx_v], z_v, sem).wait()
    pltpu.sync_copy(a_v, a_out_hbm.at[pl.ds(base, _PER_W)])
    pltpu.sync_copy(z_v, z_out_hbm.at[pl.ds(base, _PER_W)])


@functools.lru_cache(maxsize=1)
def _sc_gather():
    return pl.kernel(
        _sc_gather_body,
        out_type=[jax.ShapeDtypeStruct((_TOT,), jnp.float32),
                  jax.ShapeDtypeStruct((_TOT,), jnp.float32)],
        mesh=plsc.VectorSubcoreMesh(core_axis_name="c", subcore_axis_name="s"),
        scratch_types=[
            pltpu.VMEM((_PER_W,), jnp.int32),
            pltpu.VMEM((_PER_W,), jnp.float32),
            pltpu.VMEM((_PER_W,), jnp.float32),
            pltpu.SemaphoreType.DMA,
        ],
    )


def _tc_body(a_ref, z_ref, d_ref, o_ref):
    a = a_ref[0]
    z = z_ref[0]
    d = d_ref[0]
    ac = jnp.reshape(a, (_N, 1))
    zc = jnp.reshape(z, (_N, 1))
    alpha = ac * a
    zz = zc * z
    r = lax.rsqrt(d)
    d15 = d * d * r
    inv = r * r
    e = jnp.exp(-alpha * d15) * zz * inv
    o_ref[0] = jnp.sum(e, axis=(0, 1), keepdims=True)


def _bzz(b):
    z = jnp.int32(0)
    return (b, z, z)


def kernel(numbers, d_ij, weight):
    nums = numbers.reshape(-1).astype(jnp.int32)
    w = weight.astype(jnp.float32)
    a_tab = jnp.pad(w[:, 0], (0, _TAB - w.shape[0]))
    z_tab = jnp.pad(w[:, 1], (0, _TAB - w.shape[0]))

    a_g, z_g = _sc_gather()(nums, a_tab, z_tab)
    a_g = a_g.reshape(_B, 1, _N)
    z_g = z_g.reshape(_B, 1, _N)

    out = pl.pallas_call(
        _tc_body,
        grid=(_B,),
        in_specs=[
            pl.BlockSpec((1, 1, _N), _bzz),
            pl.BlockSpec((1, 1, _N), _bzz),
            pl.BlockSpec((1, _N, _N), _bzz),
        ],
        out_specs=pl.BlockSpec((1, 1, 1), _bzz),
        out_shape=jax.ShapeDtypeStruct((_B, 1, 1), jnp.float32),
        compiler_params=pltpu.CompilerParams(
            dimension_semantics=("arbitrary",),
        ),
    )(a_g, z_g, d_ij)

    return out.reshape(_B).astype(jnp.float64)


# speedup vs baseline: 4.5390x; 4.3724x over previous
import functools

import jax
import jax.numpy as jnp
from jax import lax
from jax.experimental import pallas as pl
from jax.experimental.pallas import tpu as pltpu
from jax.experimental.pallas import tpu_sc as plsc

_B = 64
_N = 512
_TOT = _B * _N
_NW = 32
_PER_W = _TOT // _NW
_LANES = 16
_TAB = 128


_NCHUNK = _TAB // _LANES


def _sc_gather_body(nums_hbm, a_tab_hbm, z_tab_hbm, a_out_hbm, z_out_hbm,
                    idx_v, a_v, z_v, a_tab_v, z_tab_v):
    wid = lax.axis_index("s") * 2 + lax.axis_index("c")
    base = wid * _PER_W
    pltpu.sync_copy(a_tab_hbm, a_tab_v)
    pltpu.sync_copy(z_tab_hbm, z_tab_v)
    pltpu.sync_copy(nums_hbm.at[pl.ds(base, _PER_W)], idx_v)

    def body(i, carry):
        off = i * jnp.int32(_LANES)
        idx = idx_v[pl.ds(off, _LANES)]
        lo = lax.bitwise_and(idx, jnp.int32(_LANES - 1))
        hi = lax.shift_right_logical(idx, jnp.int32(4))
        acc_a = jnp.zeros((_LANES,), jnp.float32)
        acc_z = jnp.zeros((_LANES,), jnp.float32)
        for k in range(_NCHUNK):
            ch_a = a_tab_v[pl.ds(k * _LANES, _LANES)]
            ch_z = z_tab_v[pl.ds(k * _LANES, _LANES)]
            ga = ch_a.at[lo].get(mode="promise_in_bounds")
            gz = ch_z.at[lo].get(mode="promise_in_bounds")
            m = hi == jnp.int32(k)
            acc_a = jnp.where(m, ga, acc_a)
            acc_z = jnp.where(m, gz, acc_z)
        a_v[pl.ds(off, _LANES)] = acc_a
        z_v[pl.ds(off, _LANES)] = acc_z
        return carry

    lax.fori_loop(jnp.int32(0), jnp.int32(_PER_W // _LANES), body,
                  jnp.int32(0))
    pltpu.sync_copy(a_v, a_out_hbm.at[pl.ds(base, _PER_W)])
    pltpu.sync_copy(z_v, z_out_hbm.at[pl.ds(base, _PER_W)])


@functools.lru_cache(maxsize=1)
def _sc_gather():
    return pl.kernel(
        _sc_gather_body,
        out_type=[jax.ShapeDtypeStruct((_TOT,), jnp.float32),
                  jax.ShapeDtypeStruct((_TOT,), jnp.float32)],
        mesh=plsc.VectorSubcoreMesh(core_axis_name="c", subcore_axis_name="s"),
        scratch_types=[
            pltpu.VMEM((_PER_W,), jnp.int32),
            pltpu.VMEM((_PER_W,), jnp.float32),
            pltpu.VMEM((_PER_W,), jnp.float32),
            pltpu.VMEM((_TAB,), jnp.float32),
            pltpu.VMEM((_TAB,), jnp.float32),
        ],
    )


def _tc_body(a_ref, z_ref, d_ref, o_ref):
    a = a_ref[0]
    z = z_ref[0]
    d = d_ref[0]
    ac = jnp.reshape(a, (_N, 1))
    zc = jnp.reshape(z, (_N, 1))
    alpha = ac * a
    zz = zc * z
    r = lax.rsqrt(d)
    d15 = d * d * r
    inv = r * r
    e = jnp.exp(-alpha * d15) * zz * inv
    o_ref[0] = jnp.sum(e, axis=(0, 1), keepdims=True)


def _bzz(b):
    z = jnp.int32(0)
    return (b, z, z)


def kernel(numbers, d_ij, weight):
    nums = numbers.reshape(-1).astype(jnp.int32)
    w = weight.astype(jnp.float32)
    a_tab = jnp.pad(w[:, 0], (0, _TAB - w.shape[0]))
    z_tab = jnp.pad(w[:, 1], (0, _TAB - w.shape[0]))

    a_g, z_g = _sc_gather()(nums, a_tab, z_tab)
    a_g = a_g.reshape(_B, 1, _N)
    z_g = z_g.reshape(_B, 1, _N)

    out = pl.pallas_call(
        _tc_body,
        grid=(_B,),
        in_specs=[
            pl.BlockSpec((1, 1, _N), _bzz),
            pl.BlockSpec((1, 1, _N), _bzz),
            pl.BlockSpec((1, _N, _N), _bzz),
        ],
        out_specs=pl.BlockSpec((1, 1, 1), _bzz),
        out_shape=jax.ShapeDtypeStruct((_B, 1, 1), jnp.float32),
        compiler_params=pltpu.CompilerParams(
            dimension_semantics=("arbitrary",),
        ),
    )(a_g, z_g, d_ij)

    return out.reshape(_B).astype(jnp.float64)


# speedup vs baseline: 5.6699x; 1.2491x over previous
import functools

import jax
import jax.numpy as jnp
from jax import lax
from jax.experimental import pallas as pl
from jax.experimental.pallas import tpu as pltpu
from jax.experimental.pallas import tpu_sc as plsc

_B = 64
_N = 512
_TOT = _B * _N
_NW = 32
_PER_W = _TOT // _NW
_LANES = 16
_TAB = 128


_NCHUNK = _TAB // _LANES


def _sc_gather_body(nums_hbm, a_tab_hbm, z_tab_hbm, a_out_hbm, z_out_hbm,
                    idx_v, a_v, z_v, a_tab_v, z_tab_v):
    wid = lax.axis_index("s") * 2 + lax.axis_index("c")
    base = wid * _PER_W
    pltpu.sync_copy(a_tab_hbm, a_tab_v)
    pltpu.sync_copy(z_tab_hbm, z_tab_v)
    pltpu.sync_copy(nums_hbm.at[pl.ds(base, _PER_W)], idx_v)

    def body(i, carry):
        off = i * jnp.int32(_LANES)
        idx = idx_v[pl.ds(off, _LANES)]
        lo = lax.bitwise_and(idx, jnp.int32(_LANES - 1))
        hi = lax.shift_right_logical(idx, jnp.int32(4))
        acc_a = jnp.zeros((_LANES,), jnp.float32)
        acc_z = jnp.zeros((_LANES,), jnp.float32)
        for k in range(_NCHUNK):
            ch_a = a_tab_v[pl.ds(k * _LANES, _LANES)]
            ch_z = z_tab_v[pl.ds(k * _LANES, _LANES)]
            ga = ch_a.at[lo].get(mode="promise_in_bounds")
            gz = ch_z.at[lo].get(mode="promise_in_bounds")
            m = hi == jnp.int32(k)
            acc_a = jnp.where(m, ga, acc_a)
            acc_z = jnp.where(m, gz, acc_z)
        a_v[pl.ds(off, _LANES)] = acc_a
        z_v[pl.ds(off, _LANES)] = acc_z
        return carry

    lax.fori_loop(jnp.int32(0), jnp.int32(_PER_W // _LANES), body,
                  jnp.int32(0))
    pltpu.sync_copy(a_v, a_out_hbm.at[pl.ds(base, _PER_W)])
    pltpu.sync_copy(z_v, z_out_hbm.at[pl.ds(base, _PER_W)])


@functools.lru_cache(maxsize=1)
def _sc_gather():
    return pl.kernel(
        _sc_gather_body,
        out_type=[jax.ShapeDtypeStruct((_TOT,), jnp.float32),
                  jax.ShapeDtypeStruct((_TOT,), jnp.float32)],
        mesh=plsc.VectorSubcoreMesh(core_axis_name="c", subcore_axis_name="s"),
        scratch_types=[
            pltpu.VMEM((_PER_W,), jnp.int32),
            pltpu.VMEM((_PER_W,), jnp.float32),
            pltpu.VMEM((_PER_W,), jnp.float32),
            pltpu.VMEM((_TAB,), jnp.float32),
            pltpu.VMEM((_TAB,), jnp.float32),
        ],
    )


_BB = 2


def _tc_body(a_ref, z_ref, d_ref, o_ref):
    for t in range(_BB):
        a = a_ref[t]
        z = z_ref[t]
        d = d_ref[t]
        ac = jnp.reshape(a, (_N, 1))
        zc = jnp.reshape(z, (_N, 1))
        alpha = ac * a
        zz = zc * z
        r = lax.rsqrt(d)
        d15 = d * d * r
        inv = r * r
        e = jnp.exp(-alpha * d15) * zz * inv
        o_ref[t] = jnp.sum(e, axis=(0, 1), keepdims=True)


def _bzz(b):
    z = jnp.int32(0)
    return (b, z, z)


def kernel(numbers, d_ij, weight):
    nums = numbers.reshape(-1).astype(jnp.int32)
    w = weight.astype(jnp.float32)
    a_tab = jnp.pad(w[:, 0], (0, _TAB - w.shape[0]))
    z_tab = jnp.pad(w[:, 1], (0, _TAB - w.shape[0]))

    a_g, z_g = _sc_gather()(nums, a_tab, z_tab)
    a_g = a_g.reshape(_B, 1, _N)
    z_g = z_g.reshape(_B, 1, _N)

    out = pl.pallas_call(
        _tc_body,
        grid=(_B // _BB,),
        in_specs=[
            pl.BlockSpec((_BB, 1, _N), _bzz),
            pl.BlockSpec((_BB, 1, _N), _bzz),
            pl.BlockSpec((_BB, _N, _N), _bzz),
        ],
        out_specs=pl.BlockSpec((_BB, 1, 1), _bzz),
        out_shape=jax.ShapeDtypeStruct((_B, 1, 1), jnp.float32),
        compiler_params=pltpu.CompilerParams(
            dimension_semantics=("arbitrary",),
        ),
    )(a_g, z_g, d_ij)

    return out.reshape(_B).astype(jnp.float64)


# speedup vs baseline: 6.3742x; 1.1242x over previous
import functools

import jax
import jax.numpy as jnp
from jax import lax
from jax.experimental import pallas as pl
from jax.experimental.pallas import tpu as pltpu
from jax.experimental.pallas import tpu_sc as plsc

_B = 64
_N = 512
_TOT = _B * _N
_NW = 32
_PER_W = _TOT // _NW
_LANES = 16
_TAB = 128


_NCHUNK = _TAB // _LANES


def _sc_gather_body(nums_hbm, a_tab_hbm, z_tab_hbm, a_out_hbm, z_out_hbm,
                    idx_v, a_v, z_v, a_tab_v, z_tab_v):
    wid = lax.axis_index("s") * 2 + lax.axis_index("c")
    base = wid * _PER_W
    pltpu.sync_copy(a_tab_hbm, a_tab_v)
    pltpu.sync_copy(z_tab_hbm, z_tab_v)
    pltpu.sync_copy(nums_hbm.at[pl.ds(base, _PER_W)], idx_v)

    def body(i, carry):
        off = i * jnp.int32(_LANES)
        idx = idx_v[pl.ds(off, _LANES)]
        lo = lax.bitwise_and(idx, jnp.int32(_LANES - 1))
        hi = lax.shift_right_logical(idx, jnp.int32(4))
        acc_a = jnp.zeros((_LANES,), jnp.float32)
        acc_z = jnp.zeros((_LANES,), jnp.float32)
        for k in range(_NCHUNK):
            ch_a = a_tab_v[pl.ds(k * _LANES, _LANES)]
            ch_z = z_tab_v[pl.ds(k * _LANES, _LANES)]
            ga = ch_a.at[lo].get(mode="promise_in_bounds")
            gz = ch_z.at[lo].get(mode="promise_in_bounds")
            m = hi == jnp.int32(k)
            acc_a = jnp.where(m, ga, acc_a)
            acc_z = jnp.where(m, gz, acc_z)
        a_v[pl.ds(off, _LANES)] = acc_a
        z_v[pl.ds(off, _LANES)] = acc_z
        return carry

    lax.fori_loop(jnp.int32(0), jnp.int32(_PER_W // _LANES), body,
                  jnp.int32(0))
    pltpu.sync_copy(a_v, a_out_hbm.at[pl.ds(base, _PER_W)])
    pltpu.sync_copy(z_v, z_out_hbm.at[pl.ds(base, _PER_W)])


@functools.lru_cache(maxsize=1)
def _sc_gather():
    return pl.kernel(
        _sc_gather_body,
        out_type=[jax.ShapeDtypeStruct((_TOT,), jnp.float32),
                  jax.ShapeDtypeStruct((_TOT,), jnp.float32)],
        mesh=plsc.VectorSubcoreMesh(core_axis_name="c", subcore_axis_name="s"),
        scratch_types=[
            pltpu.VMEM((_PER_W,), jnp.int32),
            pltpu.VMEM((_PER_W,), jnp.float32),
            pltpu.VMEM((_PER_W,), jnp.float32),
            pltpu.VMEM((_TAB,), jnp.float32),
            pltpu.VMEM((_TAB,), jnp.float32),
        ],
    )


_BB = 4


def _tc_body(a_ref, z_ref, d_ref, o_ref):
    for t in range(_BB):
        a = a_ref[t]
        z = z_ref[t]
        d = d_ref[t]
        ac = jnp.reshape(a, (_N, 1))
        zc = jnp.reshape(z, (_N, 1))
        alpha = ac * a
        zz = zc * z
        r = lax.rsqrt(d)
        d15 = d * d * r
        inv = r * r
        e = jnp.exp(-alpha * d15) * zz * inv
        o_ref[t] = jnp.sum(e, axis=(0, 1), keepdims=True)


def _bzz(b):
    z = jnp.int32(0)
    return (b, z, z)


def kernel(numbers, d_ij, weight):
    nums = numbers.reshape(-1).astype(jnp.int32)
    w = weight.astype(jnp.float32)
    a_tab = jnp.pad(w[:, 0], (0, _TAB - w.shape[0]))
    z_tab = jnp.pad(w[:, 1], (0, _TAB - w.shape[0]))

    a_g, z_g = _sc_gather()(nums, a_tab, z_tab)
    a_g = a_g.reshape(_B, 1, _N)
    z_g = z_g.reshape(_B, 1, _N)

    out = pl.pallas_call(
        _tc_body,
        grid=(_B // _BB,),
        in_specs=[
            pl.BlockSpec((_BB, 1, _N), _bzz),
            pl.BlockSpec((_BB, 1, _N), _bzz),
            pl.BlockSpec((_BB, _N, _N), _bzz),
        ],
        out_specs=pl.BlockSpec((_BB, 1, 1), _bzz),
        out_shape=jax.ShapeDtypeStruct((_B, 1, 1), jnp.float32),
        compiler_params=pltpu.CompilerParams(
            dimension_semantics=("arbitrary",),
        ),
    )(a_g, z_g, d_ij)

    return out.reshape(_B).astype(jnp.float64)


# speedup vs baseline: 6.3976x; 1.0037x over previous
import functools

import jax
import jax.numpy as jnp
from jax import lax
from jax.experimental import pallas as pl
from jax.experimental.pallas import tpu as pltpu
from jax.experimental.pallas import tpu_sc as plsc

_B = 64
_N = 512
_TOT = _B * _N
_NW = 32
_PER_W = _TOT // _NW
_LANES = 16
_TAB = 128


_NCHUNK = _TAB // _LANES


def _sc_gather_body(nums_hbm, a_tab_hbm, z_tab_hbm, a_out_hbm, z_out_hbm,
                    idx_v, a_v, z_v, a_tab_v, z_tab_v):
    wid = lax.axis_index("s") * 2 + lax.axis_index("c")
    base = wid * _PER_W
    pltpu.sync_copy(a_tab_hbm, a_tab_v)
    pltpu.sync_copy(z_tab_hbm, z_tab_v)
    pltpu.sync_copy(nums_hbm.at[pl.ds(base, _PER_W)], idx_v)

    def body(i, carry):
        off = i * jnp.int32(_LANES)
        idx = idx_v[pl.ds(off, _LANES)]
        lo = lax.bitwise_and(idx, jnp.int32(_LANES - 1))
        hi = lax.shift_right_logical(idx, jnp.int32(4))
        acc_a = jnp.zeros((_LANES,), jnp.float32)
        acc_z = jnp.zeros((_LANES,), jnp.float32)
        for k in range(_NCHUNK):
            ch_a = a_tab_v[pl.ds(k * _LANES, _LANES)]
            ch_z = z_tab_v[pl.ds(k * _LANES, _LANES)]
            ga = ch_a.at[lo].get(mode="promise_in_bounds")
            gz = ch_z.at[lo].get(mode="promise_in_bounds")
            m = hi == jnp.int32(k)
            acc_a = jnp.where(m, ga, acc_a)
            acc_z = jnp.where(m, gz, acc_z)
        a_v[pl.ds(off, _LANES)] = acc_a
        z_v[pl.ds(off, _LANES)] = acc_z
        return carry

    lax.fori_loop(jnp.int32(0), jnp.int32(_PER_W // _LANES), body,
                  jnp.int32(0))
    pltpu.sync_copy(a_v, a_out_hbm.at[pl.ds(base, _PER_W)])
    pltpu.sync_copy(z_v, z_out_hbm.at[pl.ds(base, _PER_W)])


@functools.lru_cache(maxsize=1)
def _sc_gather():
    return pl.kernel(
        _sc_gather_body,
        out_type=[jax.ShapeDtypeStruct((_TOT,), jnp.float32),
                  jax.ShapeDtypeStruct((_TOT,), jnp.float32)],
        mesh=plsc.VectorSubcoreMesh(core_axis_name="c", subcore_axis_name="s"),
        scratch_types=[
            pltpu.VMEM((_PER_W,), jnp.int32),
            pltpu.VMEM((_PER_W,), jnp.float32),
            pltpu.VMEM((_PER_W,), jnp.float32),
            pltpu.VMEM((_TAB,), jnp.float32),
            pltpu.VMEM((_TAB,), jnp.float32),
        ],
    )


_BB = 8


def _tc_body(a_ref, z_ref, d_ref, o_ref):
    for t in range(_BB):
        a = a_ref[t]
        z = z_ref[t]
        d = d_ref[t]
        ac = jnp.reshape(a, (_N, 1))
        zc = jnp.reshape(z, (_N, 1))
        alpha = ac * a
        zz = zc * z
        r = lax.rsqrt(d)
        d15 = d * d * r
        inv = r * r
        e = jnp.exp(-alpha * d15) * zz * inv
        o_ref[t] = jnp.sum(e, axis=(0, 1), keepdims=True)


def _bzz(b):
    z = jnp.int32(0)
    return (b, z, z)


def kernel(numbers, d_ij, weight):
    nums = numbers.reshape(-1).astype(jnp.int32)
    w = weight.astype(jnp.float32)
    a_tab = jnp.pad(w[:, 0], (0, _TAB - w.shape[0]))
    z_tab = jnp.pad(w[:, 1], (0, _TAB - w.shape[0]))

    a_g, z_g = _sc_gather()(nums, a_tab, z_tab)
    a_g = a_g.reshape(_B, 1, _N)
    z_g = z_g.reshape(_B, 1, _N)

    out = pl.pallas_call(
        _tc_body,
        grid=(_B // _BB,),
        in_specs=[
            pl.BlockSpec((_BB, 1, _N), _bzz),
            pl.BlockSpec((_BB, 1, _N), _bzz),
            pl.BlockSpec((_BB, _N, _N), _bzz),
        ],
        out_specs=pl.BlockSpec((_BB, 1, 1), _bzz),
        out_shape=jax.ShapeDtypeStruct((_B, 1, 1), jnp.float32),
        compiler_params=pltpu.CompilerParams(
            dimension_semantics=("arbitrary",),
        ),
    )(a_g, z_g, d_ij)

    return out.reshape(_B).astype(jnp.float64)
